# Initial kernel scaffold; baseline (speedup 1.0000x reference)
import jax
import jax.numpy as jnp
from jax.experimental import pallas as pl


def kernel(user_emb, item_emb, edge_index):
    raise NotImplementedError("write your pallas kernel here")


# speedup vs baseline: 12.6332x; 12.6332x over previous
import jax
import jax.numpy as jnp
from jax import lax
from jax.experimental import pallas as pl
from jax.experimental.pallas import tpu as pltpu
from jax.experimental.pallas import tpu_sc as plsc

_N_REAL = 50000
_N_PAD = 51200
_H = 16
_NC = 2
_NS = 16
_ROWS_PER_TILE = _N_PAD // _NS
_RCHUNK = 128
_NCHUNK = _ROWS_PER_TILE // _RCHUNK
_E_PAD = 1_638_400
_IDX_ROWS = _E_PAD // 128
_SB_ROWS = 32
_SB_PER_TILE = _IDX_ROWS // _NS // _SB_ROWS
_EPS = 1e-07
_N_LAYERS = 3


def _rsqrt_newton(a):
    i = lax.bitcast_convert_type(a, jnp.int32)
    i = jnp.int32(0x5F3759DF) - lax.shift_right_arithmetic(i, jnp.int32(1))
    r = lax.bitcast_convert_type(i, jnp.float32)
    half = a * 0.5
    for _ in range(3):
        r = r * (1.5 - half * r * r)
    return r


def _body(xs, rows, cols_plain, cols, out, y_hbm, c2_hbm, acc_sh, rbuf, cbuf,
          gbuf, abuf, c2buf, sbuf, onesb, zerosb):
    ci = lax.axis_index("c")
    tid = lax.axis_index("s")
    node_base = tid * _ROWS_PER_TILE
    sb_base = tid * (_SB_PER_TILE * _SB_ROWS)

    ones16 = jnp.ones((16,), jnp.float32)
    zeros16 = jnp.zeros((16,), jnp.float32)

    def fill_const(i, _):
        onesb[i, :] = ones16
        zerosb[i, :] = zeros16
        return 0

    lax.fori_loop(0, _RCHUNK, fill_const, 0)

    def zero_chunk(c, _):
        off = node_base + c * _RCHUNK
        pltpu.sync_copy(zerosb, acc_sh.at[pl.ds(off, _RCHUNK)])
        return 0

    lax.fori_loop(0, _NCHUNK, zero_chunk, 0)
    plsc.subcore_barrier()

    def deg_sb(sb, _):
        idx_off = sb_base + sb * _SB_ROWS
        pltpu.sync_copy(rows.at[pl.ds(idx_off, _SB_ROWS)], rbuf)
        pltpu.sync_copy(cols_plain.at[pl.ds(idx_off, _SB_ROWS)], cbuf)

        def deg_row(j, _):
            pltpu.sync_copy(onesb, acc_sh.at[rbuf.at[j]], add=True)
            pltpu.sync_copy(onesb, acc_sh.at[cbuf.at[j]], add=True)
            return 0

        lax.fori_loop(0, _SB_ROWS, deg_row, 0)
        return 0

    lax.fori_loop(0, _SB_PER_TILE, deg_sb, 0)
    plsc.subcore_barrier()

    def init_chunk(c, _):
        off = node_base + c * _RCHUNK
        pltpu.sync_copy(acc_sh.at[pl.ds(off, _RCHUNK)], abuf)
        pltpu.sync_copy(xs.at[ci, pl.ds(off, _RCHUNK)], gbuf)
        pltpu.sync_copy(zerosb, acc_sh.at[pl.ds(off, _RCHUNK)])

        def init_row(r, _):
            d = abuf[r, :] + _EPS
            c2 = 1.0 / d
            cc = c2 * _rsqrt_newton(c2)
            y0 = gbuf[r, :] * cc
            c2buf[r, :] = c2
            sbuf[r, :] = y0
            abuf[r, :] = y0
            return 0

        lax.fori_loop(0, _RCHUNK, init_row, 0)
        pltpu.sync_copy(c2buf, c2_hbm.at[ci, pl.ds(off, _RCHUNK)])
        pltpu.sync_copy(sbuf, out.at[ci, pl.ds(off, _RCHUNK)])
        pltpu.sync_copy(abuf, y_hbm.at[pl.ds(ci * _N_PAD + off, _RCHUNK)])
        return 0

    lax.fori_loop(0, _NCHUNK, init_chunk, 0)
    plsc.subcore_barrier()

    for layer in range(_N_LAYERS):
        last = layer == _N_LAYERS - 1

        def edge_sb(sb, _):
            idx_off = sb_base + sb * _SB_ROWS
            pltpu.sync_copy(rows.at[pl.ds(idx_off, _SB_ROWS)], rbuf)
            pltpu.sync_copy(cols.at[ci, pl.ds(idx_off, _SB_ROWS)], cbuf)

            def edge_row(j, _):
                pltpu.sync_copy(y_hbm.at[cbuf.at[j]], gbuf)
                pltpu.sync_copy(gbuf, acc_sh.at[rbuf.at[j]], add=True)
                return 0

            lax.fori_loop(0, _SB_ROWS, edge_row, 0)
            return 0

        lax.fori_loop(0, _SB_PER_TILE, edge_sb, 0)
        plsc.subcore_barrier()

        def rescale_chunk(c, _):
            off = node_base + c * _RCHUNK
            pltpu.sync_copy(acc_sh.at[pl.ds(off, _RCHUNK)], abuf)
            pltpu.sync_copy(c2_hbm.at[ci, pl.ds(off, _RCHUNK)], c2buf)
            pltpu.sync_copy(out.at[ci, pl.ds(off, _RCHUNK)], sbuf)
            if not last:
                pltpu.sync_copy(zerosb, acc_sh.at[pl.ds(off, _RCHUNK)])

            def rescale_row(r, _):
                c2 = c2buf[r, :]
                val = abuf[r, :] * c2
                s = sbuf[r, :] + val
                if last:
                    sbuf[r, :] = s * _rsqrt_newton(c2) * 0.25
                else:
                    sbuf[r, :] = s
                    abuf[r, :] = val
                return 0

            lax.fori_loop(0, _RCHUNK, rescale_row, 0)
            pltpu.sync_copy(sbuf, out.at[ci, pl.ds(off, _RCHUNK)])
            if not last:
                pltpu.sync_copy(
                    abuf, y_hbm.at[pl.ds(ci * _N_PAD + off, _RCHUNK)])
            return 0

        lax.fori_loop(0, _NCHUNK, rescale_chunk, 0)
        if not last:
            plsc.subcore_barrier()


@jax.jit
def _lightgcn(xs, rows, cols_plain, cols):
    mesh = plsc.VectorSubcoreMesh(core_axis_name="c", subcore_axis_name="s")
    out, _, _ = pl.kernel(
        _body,
        out_type=(
            jax.ShapeDtypeStruct((_NC, _N_PAD, _H), jnp.float32),
            jax.ShapeDtypeStruct((_NC * _N_PAD, _H), jnp.float32),
            jax.ShapeDtypeStruct((_NC, _N_PAD, _H), jnp.float32),
        ),
        mesh=mesh,
        compiler_params=pltpu.CompilerParams(use_tc_tiling_on_sc=False),
        scratch_types=[
            pltpu.VMEM_SHARED((_N_PAD, _H), jnp.float32),
            pltpu.VMEM((_SB_ROWS, 128), jnp.int32),
            pltpu.VMEM((_SB_ROWS, 128), jnp.int32),
            pltpu.VMEM((_RCHUNK, _H), jnp.float32),
            pltpu.VMEM((_RCHUNK, _H), jnp.float32),
            pltpu.VMEM((_RCHUNK, _H), jnp.float32),
            pltpu.VMEM((_RCHUNK, _H), jnp.float32),
            pltpu.VMEM((_RCHUNK, _H), jnp.float32),
            pltpu.VMEM((_RCHUNK, _H), jnp.float32),
        ],
    )(xs, rows, cols_plain, cols)
    return out


def kernel(user_emb, item_emb, edge_index):
    n_users = user_emb.shape[0]
    n_items = item_emb.shape[0]
    ego = jnp.concatenate([user_emb, item_emb], axis=0)
    ego = jnp.pad(ego, ((0, _N_PAD - _N_REAL), (0, 0)))
    xs = ego.reshape(_N_PAD, _NC, _H).transpose(1, 0, 2)

    n_edges = edge_index.shape[1]
    pad = _E_PAD - n_edges
    dummy = jnp.full((pad,), _N_REAL, jnp.int32)
    rows = jnp.concatenate([edge_index[0], dummy]).reshape(_IDX_ROWS, 128)
    cols_plain = jnp.concatenate([edge_index[1], dummy]).reshape(_IDX_ROWS, 128)
    cols = jnp.stack([cols_plain, cols_plain + _N_PAD])

    out = _lightgcn(xs, rows, cols_plain, cols)
    full = out.transpose(1, 0, 2).reshape(_N_PAD, _NC * _H)
    return (full[:n_users], full[n_users:n_users + n_items])


# speedup vs baseline: 20.8382x; 1.6495x over previous
import jax
import jax.numpy as jnp
from jax import lax
from jax.experimental import pallas as pl
from jax.experimental.pallas import tpu as pltpu
from jax.experimental.pallas import tpu_sc as plsc

_N_REAL = 50000
_N_PAD = 51200
_H = 16
_NC = 2
_NS = 16
_ROWS_PER_TILE = _N_PAD // _NS
_RCHUNK = 400
_NCHUNK = _ROWS_PER_TILE // _RCHUNK
_E_PAD = 1_638_400
_IDX_ROWS = _E_PAD // 128
_SB_ROWS = 32
_SB_PER_TILE = _IDX_ROWS // _NS // _SB_ROWS
_G = 8
_GROUPS = _SB_ROWS // _G
_EPS = 1e-07
_N_LAYERS = 3


def _rsqrt_newton(a):
    i = lax.bitcast_convert_type(a, jnp.int32)
    i = jnp.int32(0x5F3759DF) - lax.shift_right_arithmetic(i, jnp.int32(1))
    r = lax.bitcast_convert_type(i, jnp.float32)
    half = a * 0.5
    for _ in range(3):
        r = r * (1.5 - half * r * r)
    return r


def _body(xs, rows, cols_plain, cols, out, y_hbm, c2_hbm, acc_sh,
          rbuf, cbuf, gring, abuf, gbuf, c2buf, sbuf, onesb, zerosb,
          gsem, ssem):
    ci = lax.axis_index("c")
    tid = lax.axis_index("s")
    node_base = tid * _ROWS_PER_TILE
    sb_base = tid * (_SB_PER_TILE * _SB_ROWS)

    ones16 = jnp.ones((16,), jnp.float32)
    zeros16 = jnp.zeros((16,), jnp.float32)

    def fill_ones(i, _):
        onesb[i, :] = ones16
        return 0

    def fill_zeros(i, _):
        zerosb[i, :] = zeros16
        return 0

    lax.fori_loop(0, 128, fill_ones, 0)
    lax.fori_loop(0, _RCHUNK, fill_zeros, 0)

    def zero_chunk(c, _):
        off = node_base + c * _RCHUNK
        pltpu.sync_copy(zerosb, acc_sh.at[pl.ds(off, _RCHUNK)])
        return 0

    lax.fori_loop(0, _NCHUNK, zero_chunk, 0)
    plsc.subcore_barrier()

    def deg_sb(sb, _):
        idx_off = sb_base + sb * _SB_ROWS
        pltpu.sync_copy(rows.at[pl.ds(idx_off, _SB_ROWS)], rbuf)
        pltpu.sync_copy(cols_plain.at[pl.ds(idx_off, _SB_ROWS)], cbuf)

        def deg_grp(grp, _):
            def fire(i, _):
                j = grp * _G + i
                pltpu.async_copy(onesb, acc_sh.at[rbuf.at[j]], ssem,
                                 add=True)
                pltpu.async_copy(onesb, acc_sh.at[cbuf.at[j]], ssem,
                                 add=True)
                return 0

            def drain(i, _):
                j = grp * _G + i
                pltpu.make_async_copy(
                    onesb, acc_sh.at[rbuf.at[j]], ssem).wait()
                pltpu.make_async_copy(
                    onesb, acc_sh.at[cbuf.at[j]], ssem).wait()
                return 0

            lax.fori_loop(0, _G, fire, 0)
            lax.fori_loop(0, _G, drain, 0)
            return 0

        lax.fori_loop(0, _GROUPS, deg_grp, 0)
        return 0

    lax.fori_loop(0, _SB_PER_TILE, deg_sb, 0)
    plsc.subcore_barrier()

    def init_chunk(c, _):
        off = node_base + c * _RCHUNK
        pltpu.sync_copy(acc_sh.at[pl.ds(off, _RCHUNK)], abuf)
        pltpu.sync_copy(xs.at[ci, pl.ds(off, _RCHUNK)], gbuf)
        pltpu.sync_copy(zerosb, acc_sh.at[pl.ds(off, _RCHUNK)])

        def init_row(r, _):
            d = abuf[r, :] + _EPS
            c2 = 1.0 / d
            cc = c2 * _rsqrt_newton(c2)
            y0 = gbuf[r, :] * cc
            c2buf[r, :] = c2
            sbuf[r, :] = y0
            abuf[r, :] = y0
            return 0

        lax.fori_loop(0, _RCHUNK, init_row, 0)
        pltpu.sync_copy(c2buf, c2_hbm.at[ci, pl.ds(off, _RCHUNK)])
        pltpu.sync_copy(sbuf, out.at[ci, pl.ds(off, _RCHUNK)])
        pltpu.sync_copy(abuf, y_hbm.at[pl.ds(ci * _N_PAD + off, _RCHUNK)])
        return 0

    lax.fori_loop(0, _NCHUNK, init_chunk, 0)
    plsc.subcore_barrier()

    def gather_j(j, half, i):
        return (y_hbm.at[cbuf.at[j]], gring.at[half, i], gsem)

    def scatter_j(j, half, i):
        return (gring.at[half, i], acc_sh.at[rbuf.at[j]], ssem)

    def edge_sb(sb, _):
        idx_off = sb_base + sb * _SB_ROWS
        pltpu.sync_copy(rows.at[pl.ds(idx_off, _SB_ROWS)], rbuf)
        pltpu.sync_copy(cols.at[ci, pl.ds(idx_off, _SB_ROWS)], cbuf)

        def fire_g(grp):
            half = lax.rem(grp, 2)

            def f(i, _):
                pltpu.async_copy(*gather_j(grp * _G + i, half, i))
                return 0

            lax.fori_loop(0, _G, f, 0)

        def drain_g(grp):
            half = lax.rem(grp, 2)

            def f(i, _):
                pltpu.make_async_copy(*gather_j(grp * _G + i, half, i)).wait()
                return 0

            lax.fori_loop(0, _G, f, 0)

        def fire_s(grp):
            half = lax.rem(grp, 2)

            def f(i, _):
                pltpu.async_copy(*scatter_j(grp * _G + i, half, i), add=True)
                return 0

            lax.fori_loop(0, _G, f, 0)

        def drain_s(grp):
            half = lax.rem(grp, 2)

            def f(i, _):
                pltpu.make_async_copy(
                    *scatter_j(grp * _G + i, half, i)).wait()
                return 0

            lax.fori_loop(0, _G, f, 0)

        fire_g(jnp.int32(0))

        def grp_body(grp, _):
            drain_g(grp)

            @pl.when(grp > 0)
            def _():
                drain_s(grp - 1)

            fire_s(grp)

            @pl.when(grp < _GROUPS - 1)
            def _():
                fire_g(grp + 1)

            return 0

        lax.fori_loop(0, _GROUPS, grp_body, 0)
        drain_s(jnp.int32(_GROUPS - 1))
        return 0

    for layer in range(_N_LAYERS):
        last = layer == _N_LAYERS - 1

        lax.fori_loop(0, _SB_PER_TILE, edge_sb, 0)
        plsc.subcore_barrier()

        def rescale_chunk(c, _):
            off = node_base + c * _RCHUNK
            pltpu.sync_copy(acc_sh.at[pl.ds(off, _RCHUNK)], abuf)
            pltpu.sync_copy(c2_hbm.at[ci, pl.ds(off, _RCHUNK)], c2buf)
            pltpu.sync_copy(out.at[ci, pl.ds(off, _RCHUNK)], sbuf)
            if not last:
                pltpu.sync_copy(zerosb, acc_sh.at[pl.ds(off, _RCHUNK)])

            def rescale_row(r, _):
                c2 = c2buf[r, :]
                val = abuf[r, :] * c2
                s = sbuf[r, :] + val
                if last:
                    sbuf[r, :] = s * _rsqrt_newton(c2) * 0.25
                else:
                    sbuf[r, :] = s
                    abuf[r, :] = val
                return 0

            lax.fori_loop(0, _RCHUNK, rescale_row, 0)
            pltpu.sync_copy(sbuf, out.at[ci, pl.ds(off, _RCHUNK)])
            if not last:
                pltpu.sync_copy(
                    abuf, y_hbm.at[pl.ds(ci * _N_PAD + off, _RCHUNK)])
            return 0

        lax.fori_loop(0, _NCHUNK, rescale_chunk, 0)
        if not last:
            plsc.subcore_barrier()


@jax.jit
def _lightgcn(xs, rows, cols_plain, cols):
    mesh = plsc.VectorSubcoreMesh(core_axis_name="c", subcore_axis_name="s")
    out, _, _ = pl.kernel(
        _body,
        out_type=(
            jax.ShapeDtypeStruct((_NC, _N_PAD, _H), jnp.float32),
            jax.ShapeDtypeStruct((_NC * _N_PAD, _H), jnp.float32),
            jax.ShapeDtypeStruct((_NC, _N_PAD, _H), jnp.float32),
        ),
        mesh=mesh,
        compiler_params=pltpu.CompilerParams(use_tc_tiling_on_sc=False),
        scratch_types=[
            pltpu.VMEM_SHARED((_N_PAD, _H), jnp.float32),
            pltpu.VMEM((_SB_ROWS, 128), jnp.int32),
            pltpu.VMEM((_SB_ROWS, 128), jnp.int32),
            pltpu.VMEM((2, _G, 128, _H), jnp.float32),
            pltpu.VMEM((_RCHUNK, _H), jnp.float32),
            pltpu.VMEM((_RCHUNK, _H), jnp.float32),
            pltpu.VMEM((_RCHUNK, _H), jnp.float32),
            pltpu.VMEM((_RCHUNK, _H), jnp.float32),
            pltpu.VMEM((128, _H), jnp.float32),
            pltpu.VMEM((_RCHUNK, _H), jnp.float32),
            pltpu.SemaphoreType.DMA,
            pltpu.SemaphoreType.DMA,
        ],
    )(xs, rows, cols_plain, cols)
    return out


def kernel(user_emb, item_emb, edge_index):
    n_users = user_emb.shape[0]
    n_items = item_emb.shape[0]
    ego = jnp.concatenate([user_emb, item_emb], axis=0)
    ego = jnp.pad(ego, ((0, _N_PAD - _N_REAL), (0, 0)))
    xs = ego.reshape(_N_PAD, _NC, _H).transpose(1, 0, 2)

    n_edges = edge_index.shape[1]
    pad = _E_PAD - n_edges
    dummy = jnp.full((pad,), _N_REAL, jnp.int32)
    rows = jnp.concatenate([edge_index[0], dummy]).reshape(_IDX_ROWS, 128)
    cols_plain = jnp.concatenate([edge_index[1], dummy]).reshape(_IDX_ROWS, 128)
    cols = jnp.stack([cols_plain, cols_plain + _N_PAD])

    out = _lightgcn(xs, rows, cols_plain, cols)
    full = out.transpose(1, 0, 2).reshape(_N_PAD, _NC * _H)
    return (full[:n_users], full[n_users:n_users + n_items])
